# baseline (speedup 1.0000x reference)
import jax
import jax.numpy as jnp
import numpy as np
from jax import lax
from jax.experimental import pallas as pl
from jax.experimental.pallas import tpu as pltpu
from jax.experimental.pallas import tpu_sc as plsc

_SEQ_LEN = 8192
_EMBED_DIM = 64
_VOCAB1 = 100001


def _pos_encoding_np(position, d_model):
    i = np.arange(d_model)[np.newaxis, :]
    pos = np.arange(position)[:, np.newaxis]
    angle_rates = 1.0 / np.power(10000, 2 * (i // 2) / np.float32(d_model))
    angle_rads = pos * angle_rates
    angle_rads[:, 0::2] = np.sin(angle_rads[:, 0::2])
    angle_rads[:, 1::2] = np.cos(angle_rads[:, 1::2])
    return angle_rads.astype(np.float32)


_POS_T = np.ascontiguousarray(_pos_encoding_np(_SEQ_LEN, _EMBED_DIM).T)

_INFO = plsc.get_sparse_core_info()
_NC, _NS = _INFO.num_cores, _INFO.num_subcores
_NW = _NC * _NS
_E_PER_W = _EMBED_DIM // _NW


def _sc_body(x_hbm, posT_hbm, tabT_hbm, outT_hbm, idx_v, row_v, acc0_v,
             acc1_v, sem_x, sem_r, sem_a, sem_b, sem_w):
    wid = lax.axis_index("s") * _NC + lax.axis_index("c")
    e0 = wid * _E_PER_W
    e1 = e0 + 1
    with jax.named_scope("stage"):
        x_cp = pltpu.async_copy(x_hbm, idx_v, sem_x)
        row_cp = pltpu.async_copy(tabT_hbm.at[e0], row_v, sem_r)
        pos0_cp = pltpu.async_copy(posT_hbm.at[e0], acc0_v, sem_a)
        x_cp.wait()
        pos0_cp.wait()
        row_cp.wait()

    def make_gat(acc_v):
        def gat(g, _):
            v16 = idx_v[pl.ds(g * 16, 16)]
            vals = plsc.load_gather(row_v, [v16])
            plsc.addupdate(acc_v.at[pl.ds(g * 16, 16)], vals)
            return 0
        return gat

    with jax.named_scope("gather0"):
        lax.fori_loop(0, _SEQ_LEN // 16, make_gat(acc0_v), 0, unroll=16)

    with jax.named_scope("stage1"):
        wb0_cp = pltpu.async_copy(acc0_v, outT_hbm.at[e0], sem_w)
        row1_cp = pltpu.async_copy(tabT_hbm.at[e1], row_v, sem_r)
        pos1_cp = pltpu.async_copy(posT_hbm.at[e1], acc1_v, sem_b)
        pos1_cp.wait()
        row1_cp.wait()

    with jax.named_scope("gather1"):
        half = _SEQ_LEN // 32
        lax.fori_loop(0, half, make_gat(acc1_v), 0, unroll=16)
        wb1a_cp = pltpu.async_copy(acc1_v.at[pl.ds(0, _SEQ_LEN // 2)],
                                   outT_hbm.at[e1, pl.ds(0, _SEQ_LEN // 2)],
                                   sem_w)
        lax.fori_loop(half, 2 * half, make_gat(acc1_v), 0, unroll=16)

    with jax.named_scope("writeback"):
        wb0_cp.wait()
        wb1a_cp.wait()
        pltpu.sync_copy(acc1_v.at[pl.ds(_SEQ_LEN // 2, _SEQ_LEN // 2)],
                        outT_hbm.at[e1, pl.ds(_SEQ_LEN // 2, _SEQ_LEN // 2)])


def _embed(x_i32, posT, tabT):
    mesh = plsc.VectorSubcoreMesh(core_axis_name="c", subcore_axis_name="s")
    return pl.kernel(
        _sc_body,
        out_type=jax.ShapeDtypeStruct((_EMBED_DIM, _SEQ_LEN), jnp.float32),
        mesh=mesh,
        scratch_types=[
            pltpu.VMEM((_SEQ_LEN,), jnp.int32),
            pltpu.VMEM((_VOCAB1,), jnp.float32),
            pltpu.VMEM((_SEQ_LEN,), jnp.float32),
            pltpu.VMEM((_SEQ_LEN,), jnp.float32),
            pltpu.SemaphoreType.DMA,
            pltpu.SemaphoreType.DMA,
            pltpu.SemaphoreType.DMA,
            pltpu.SemaphoreType.DMA,
            pltpu.SemaphoreType.DMA,
        ],
        compiler_params=pltpu.CompilerParams(use_tc_tiling_on_sc=True,
                                            needs_layout_passes=False),
    )(x_i32, posT, tabT)


def kernel(x, table):
    x_i32 = x.astype(jnp.int32)
    posT = jnp.asarray(_POS_T)
    outT = _embed(x_i32, posT, table.T)
    return outT.T.reshape(1, _SEQ_LEN, _EMBED_DIM)


# speedup vs baseline: 1.0055x; 1.0055x over previous
import jax
import jax.numpy as jnp
import numpy as np
from jax import lax
from jax.experimental import pallas as pl
from jax.experimental.pallas import tpu as pltpu
from jax.experimental.pallas import tpu_sc as plsc

_SEQ_LEN = 8192
_EMBED_DIM = 64
_VOCAB1 = 100001


def _pos_encoding_np(position, d_model):
    i = np.arange(d_model)[np.newaxis, :]
    pos = np.arange(position)[:, np.newaxis]
    angle_rates = 1.0 / np.power(10000, 2 * (i // 2) / np.float32(d_model))
    angle_rads = pos * angle_rates
    angle_rads[:, 0::2] = np.sin(angle_rads[:, 0::2])
    angle_rads[:, 1::2] = np.cos(angle_rads[:, 1::2])
    return angle_rads.astype(np.float32)


_POS_T = np.ascontiguousarray(_pos_encoding_np(_SEQ_LEN, _EMBED_DIM).T)

_INFO = plsc.get_sparse_core_info()
_NC, _NS = _INFO.num_cores, _INFO.num_subcores
_NW = _NC * _NS
_E_PER_W = _EMBED_DIM // _NW


def _sc_body(x_hbm, posT_hbm, tabT_hbm, outT_hbm, idx_v, row_v, acc0_v,
             acc1_v, sem_x, sem_r, sem_a, sem_b, sem_w):
    wid = lax.axis_index("s") * _NC + lax.axis_index("c")
    e0 = wid * _E_PER_W
    e1 = e0 + 1
    with jax.named_scope("stage"):
        x_cp = pltpu.async_copy(x_hbm, idx_v, sem_x)
        row_cp = pltpu.async_copy(tabT_hbm.at[e0], row_v, sem_r)
        pos0_cp = pltpu.async_copy(posT_hbm.at[e0], acc0_v, sem_a)
        x_cp.wait()
        pos0_cp.wait()
        row_cp.wait()

    def make_gat(acc_v):
        def gat(g, _):
            v16 = idx_v[pl.ds(g * 16, 16)]
            vals = plsc.load_gather(row_v, [v16])
            plsc.addupdate(acc_v.at[pl.ds(g * 16, 16)], vals)
            return 0
        return gat

    with jax.named_scope("gather0"):
        lax.fori_loop(0, _SEQ_LEN // 16, make_gat(acc0_v), 0, unroll=8)

    with jax.named_scope("stage1"):
        wb0_cp = pltpu.async_copy(acc0_v, outT_hbm.at[e0], sem_w)
        row1_cp = pltpu.async_copy(tabT_hbm.at[e1], row_v, sem_r)
        pos1_cp = pltpu.async_copy(posT_hbm.at[e1], acc1_v, sem_b)
        pos1_cp.wait()
        row1_cp.wait()

    with jax.named_scope("gather1"):
        half = _SEQ_LEN // 32
        lax.fori_loop(0, half, make_gat(acc1_v), 0, unroll=8)
        wb1a_cp = pltpu.async_copy(acc1_v.at[pl.ds(0, _SEQ_LEN // 2)],
                                   outT_hbm.at[e1, pl.ds(0, _SEQ_LEN // 2)],
                                   sem_w)
        lax.fori_loop(half, 2 * half, make_gat(acc1_v), 0, unroll=8)

    with jax.named_scope("writeback"):
        wb0_cp.wait()
        wb1a_cp.wait()
        pltpu.sync_copy(acc1_v.at[pl.ds(_SEQ_LEN // 2, _SEQ_LEN // 2)],
                        outT_hbm.at[e1, pl.ds(_SEQ_LEN // 2, _SEQ_LEN // 2)])


def _embed(x_i32, posT, tabT):
    mesh = plsc.VectorSubcoreMesh(core_axis_name="c", subcore_axis_name="s")
    return pl.kernel(
        _sc_body,
        out_type=jax.ShapeDtypeStruct((_EMBED_DIM, _SEQ_LEN), jnp.float32),
        mesh=mesh,
        scratch_types=[
            pltpu.VMEM((_SEQ_LEN,), jnp.int32),
            pltpu.VMEM((_VOCAB1,), jnp.float32),
            pltpu.VMEM((_SEQ_LEN,), jnp.float32),
            pltpu.VMEM((_SEQ_LEN,), jnp.float32),
            pltpu.SemaphoreType.DMA,
            pltpu.SemaphoreType.DMA,
            pltpu.SemaphoreType.DMA,
            pltpu.SemaphoreType.DMA,
            pltpu.SemaphoreType.DMA,
        ],
        compiler_params=pltpu.CompilerParams(use_tc_tiling_on_sc=True,
                                            needs_layout_passes=False),
    )(x_i32, posT, tabT)


def kernel(x, table):
    x_i32 = x.astype(jnp.int32)
    posT = jnp.asarray(_POS_T)
    outT = _embed(x_i32, posT, table.T)
    return outT.T.reshape(1, _SEQ_LEN, _EMBED_DIM)
